# Initial kernel scaffold; baseline (speedup 1.0000x reference)
import jax
import jax.numpy as jnp
from jax.experimental import pallas as pl


def kernel(sentence, word_table, tag_table):
    raise NotImplementedError("write your pallas kernel here")


# speedup vs baseline: 1.4938x; 1.4938x over previous
import functools

import jax
import jax.numpy as jnp
from jax import lax
from jax.experimental import pallas as pl
from jax.experimental.pallas import tpu as pltpu
from jax.experimental.pallas import tpu_sc as plsc

NC, NS, LANES = 2, 16, 16
NW = NC * NS
SUB = 128


def _body(n_tok, tags, wdim, tdim,
          sent_hbm, wtab_hbm, ttab_hbm, out_hbm,
          idx_v, tag_v, w_v, t_v, sem_w, sem_t):
    tok_per_w = n_tok // NW
    nsub = tok_per_w // SUB
    wid = lax.axis_index("s") * NC + lax.axis_index("c")
    base_w = wid * tok_per_w

    def step(j, carry):
        base = base_w + j * SUB
        pltpu.sync_copy(sent_hbm.at[pl.ds(base, SUB)], idx_v)
        for i in range(SUB // LANES):
            sl = pl.ds(i * LANES, LANES)
            tag_v[sl] = lax.rem(idx_v[sl], jnp.int32(tags))
        cp_w = pltpu.async_copy(wtab_hbm.at[idx_v], w_v, sem_w)
        cp_t = pltpu.async_copy(ttab_hbm.at[tag_v], t_v, sem_t)
        cp_w.wait()
        cp_t.wait()
        pltpu.sync_copy(w_v, out_hbm.at[pl.ds(base, SUB), pl.ds(0, wdim)])
        pltpu.sync_copy(t_v, out_hbm.at[pl.ds(base, SUB), pl.ds(wdim, tdim)])
        return carry

    lax.fori_loop(0, nsub, step, 0)


def kernel(sentence, word_table, tag_table):
    b, l = sentence.shape
    n_tok = b * l
    vocab, wdim = word_table.shape
    tags, tdim = tag_table.shape
    odim = wdim + tdim
    sent = sentence.reshape(n_tok).astype(jnp.int32)

    mesh = plsc.VectorSubcoreMesh(
        core_axis_name="c", subcore_axis_name="s",
        num_cores=NC, num_subcores=NS)
    run = pl.kernel(
        functools.partial(_body, n_tok, tags, wdim, tdim),
        out_type=jax.ShapeDtypeStruct((n_tok, odim), jnp.float32),
        mesh=mesh,
        scratch_types=[
            pltpu.VMEM((SUB,), jnp.int32),
            pltpu.VMEM((SUB,), jnp.int32),
            pltpu.VMEM((SUB, wdim), jnp.float32),
            pltpu.VMEM((SUB, tdim), jnp.float32),
            pltpu.SemaphoreType.DMA,
            pltpu.SemaphoreType.DMA,
        ],
        compiler_params=pltpu.CompilerParams(use_tc_tiling_on_sc=False),
    )
    out = run(sent, word_table, tag_table)
    return out.reshape(b, l, odim)


# speedup vs baseline: 1.5103x; 1.0110x over previous
import functools

import jax
import jax.numpy as jnp
from jax import lax
from jax.experimental import pallas as pl
from jax.experimental.pallas import tpu as pltpu
from jax.experimental.pallas import tpu_sc as plsc

NC, NS, LANES = 2, 16, 16
NW = NC * NS
IDXB = 128
NIDX = 4
MAC = IDXB * NIDX
NSLOT = 2


def _body(n_tok, tags, wdim, tdim,
          sent_hbm, wtab_hbm, ttab_hbm, out_hbm,
          idx_v, tag_v, w_v, t_v, idx_sem, gw_sem, gt_sem, out_sem):
    tok_per_w = n_tok // NW
    nmac = tok_per_w // MAC
    wid = lax.axis_index("s") * NC + lax.axis_index("c")
    base_w = wid * tok_per_w

    def idx_src(g):
        return sent_hbm.at[pl.ds((base_w + g * MAC) // IDXB, NIDX)]

    def out_w_dst(g):
        return out_hbm.at[pl.ds(base_w + g * MAC, MAC), pl.ds(0, wdim)]

    def out_t_dst(g):
        return out_hbm.at[pl.ds(base_w + g * MAC, MAC), pl.ds(wdim, tdim)]

    pltpu.async_copy(idx_src(0), idx_v[0], idx_sem[0])

    def macro(gg, carry):
        for s in range(NSLOT):
            g = gg * NSLOT + s
            pltpu.make_async_copy(idx_src(g), idx_v[s], idx_sem[s]).wait()
            for i in range(NIDX):
                for j in range(IDXB // LANES):
                    sl = pl.ds(j * LANES, LANES)
                    tag_v[s][i, sl] = lax.rem(idx_v[s][i, sl],
                                              jnp.int32(tags))
            @pl.when(gg > 0)
            def _():
                pltpu.make_async_copy(w_v[s], out_w_dst(g), out_sem[s]).wait()
                pltpu.make_async_copy(t_v[s], out_t_dst(g), out_sem[s]).wait()
            for i in range(NIDX):
                rows = pl.ds(i * IDXB, IDXB)
                pltpu.async_copy(wtab_hbm.at[idx_v[s].at[i]],
                                 w_v[s].at[rows], gw_sem[s])
                pltpu.async_copy(ttab_hbm.at[tag_v[s].at[i]],
                                 t_v[s].at[rows], gt_sem[s])
            if s == 0:
                pltpu.async_copy(idx_src(g + 1), idx_v[1], idx_sem[1])
            else:
                @pl.when(gg < nmac // NSLOT - 1)
                def _():
                    pltpu.async_copy(idx_src(g + 1), idx_v[0], idx_sem[0])
            for i in range(NIDX):
                rows = pl.ds(i * IDXB, IDXB)
                pltpu.make_async_copy(wtab_hbm.at[idx_v[s].at[i]],
                                      w_v[s].at[rows], gw_sem[s]).wait()
                pltpu.make_async_copy(ttab_hbm.at[tag_v[s].at[i]],
                                      t_v[s].at[rows], gt_sem[s]).wait()
            pltpu.async_copy(w_v[s], out_w_dst(g), out_sem[s])
            pltpu.async_copy(t_v[s], out_t_dst(g), out_sem[s])
        return carry

    lax.fori_loop(0, nmac // NSLOT, macro, 0)

    for s in range(NSLOT):
        g = nmac - NSLOT + s
        pltpu.make_async_copy(w_v[s], out_w_dst(g), out_sem[s]).wait()
        pltpu.make_async_copy(t_v[s], out_t_dst(g), out_sem[s]).wait()


def kernel(sentence, word_table, tag_table):
    b, l = sentence.shape
    n_tok = b * l
    vocab, wdim = word_table.shape
    tags, tdim = tag_table.shape
    odim = wdim + tdim
    sent = sentence.reshape(n_tok // IDXB, IDXB).astype(jnp.int32)

    mesh = plsc.VectorSubcoreMesh(
        core_axis_name="c", subcore_axis_name="s",
        num_cores=NC, num_subcores=NS)
    run = pl.kernel(
        functools.partial(_body, n_tok, tags, wdim, tdim),
        out_type=jax.ShapeDtypeStruct((n_tok, odim), jnp.float32),
        mesh=mesh,
        scratch_types=[
            [pltpu.VMEM((NIDX, IDXB), jnp.int32) for _ in range(NSLOT)],
            [pltpu.VMEM((NIDX, IDXB), jnp.int32) for _ in range(NSLOT)],
            [pltpu.VMEM((MAC, wdim), jnp.float32) for _ in range(NSLOT)],
            [pltpu.VMEM((MAC, tdim), jnp.float32) for _ in range(NSLOT)],
            [pltpu.SemaphoreType.DMA for _ in range(NSLOT)],
            [pltpu.SemaphoreType.DMA for _ in range(NSLOT)],
            [pltpu.SemaphoreType.DMA for _ in range(NSLOT)],
            [pltpu.SemaphoreType.DMA for _ in range(NSLOT)],
        ],
        compiler_params=pltpu.CompilerParams(use_tc_tiling_on_sc=False),
    )
    out = run(sent, word_table, tag_table)
    return out.reshape(b, l, odim)


# speedup vs baseline: 1.5147x; 1.0029x over previous
import functools

import jax
import jax.numpy as jnp
from jax import lax
from jax.experimental import pallas as pl
from jax.experimental.pallas import tpu as pltpu
from jax.experimental.pallas import tpu_sc as plsc

NC, NS, LANES = 2, 16, 16
NW = NC * NS
IDXB = 512
NIDX = 1
MAC = IDXB * NIDX
NSLOT = 2


def _body(n_tok, tags, wdim, tdim,
          sent_hbm, wtab_hbm, ttab_hbm, out_hbm,
          idx_v, tag_v, w_v, t_v, idx_sem, gw_sem, gt_sem, out_sem):
    tok_per_w = n_tok // NW
    nmac = tok_per_w // MAC
    wid = lax.axis_index("s") * NC + lax.axis_index("c")
    base_w = wid * tok_per_w

    def idx_src(g):
        return sent_hbm.at[pl.ds((base_w + g * MAC) // IDXB, NIDX)]

    def out_w_dst(g):
        return out_hbm.at[pl.ds(base_w + g * MAC, MAC), pl.ds(0, wdim)]

    def out_t_dst(g):
        return out_hbm.at[pl.ds(base_w + g * MAC, MAC), pl.ds(wdim, tdim)]

    pltpu.async_copy(idx_src(0), idx_v[0], idx_sem[0])

    def macro(gg, carry):
        for s in range(NSLOT):
            g = gg * NSLOT + s
            pltpu.make_async_copy(idx_src(g), idx_v[s], idx_sem[s]).wait()
            for i in range(NIDX):
                for j in range(IDXB // LANES):
                    sl = pl.ds(j * LANES, LANES)
                    tag_v[s][i, sl] = lax.rem(idx_v[s][i, sl],
                                              jnp.int32(tags))
            @pl.when(gg > 0)
            def _():
                pltpu.make_async_copy(w_v[s], out_w_dst(g), out_sem[s]).wait()
                pltpu.make_async_copy(t_v[s], out_t_dst(g), out_sem[s]).wait()
            for i in range(NIDX):
                rows = pl.ds(i * IDXB, IDXB)
                pltpu.async_copy(wtab_hbm.at[idx_v[s].at[i]],
                                 w_v[s].at[rows], gw_sem[s])
                pltpu.async_copy(ttab_hbm.at[tag_v[s].at[i]],
                                 t_v[s].at[rows], gt_sem[s])
            if s == 0:
                pltpu.async_copy(idx_src(g + 1), idx_v[1], idx_sem[1])
            else:
                @pl.when(gg < nmac // NSLOT - 1)
                def _():
                    pltpu.async_copy(idx_src(g + 1), idx_v[0], idx_sem[0])
            for i in range(NIDX):
                rows = pl.ds(i * IDXB, IDXB)
                pltpu.make_async_copy(wtab_hbm.at[idx_v[s].at[i]],
                                      w_v[s].at[rows], gw_sem[s]).wait()
                pltpu.make_async_copy(ttab_hbm.at[tag_v[s].at[i]],
                                      t_v[s].at[rows], gt_sem[s]).wait()
            pltpu.async_copy(w_v[s], out_w_dst(g), out_sem[s])
            pltpu.async_copy(t_v[s], out_t_dst(g), out_sem[s])
        return carry

    lax.fori_loop(0, nmac // NSLOT, macro, 0)

    for s in range(NSLOT):
        g = nmac - NSLOT + s
        pltpu.make_async_copy(w_v[s], out_w_dst(g), out_sem[s]).wait()
        pltpu.make_async_copy(t_v[s], out_t_dst(g), out_sem[s]).wait()


def kernel(sentence, word_table, tag_table):
    b, l = sentence.shape
    n_tok = b * l
    vocab, wdim = word_table.shape
    tags, tdim = tag_table.shape
    odim = wdim + tdim
    sent = sentence.reshape(n_tok // IDXB, IDXB).astype(jnp.int32)

    mesh = plsc.VectorSubcoreMesh(
        core_axis_name="c", subcore_axis_name="s",
        num_cores=NC, num_subcores=NS)
    run = pl.kernel(
        functools.partial(_body, n_tok, tags, wdim, tdim),
        out_type=jax.ShapeDtypeStruct((n_tok, odim), jnp.float32),
        mesh=mesh,
        scratch_types=[
            [pltpu.VMEM((NIDX, IDXB), jnp.int32) for _ in range(NSLOT)],
            [pltpu.VMEM((NIDX, IDXB), jnp.int32) for _ in range(NSLOT)],
            [pltpu.VMEM((MAC, wdim), jnp.float32) for _ in range(NSLOT)],
            [pltpu.VMEM((MAC, tdim), jnp.float32) for _ in range(NSLOT)],
            [pltpu.SemaphoreType.DMA for _ in range(NSLOT)],
            [pltpu.SemaphoreType.DMA for _ in range(NSLOT)],
            [pltpu.SemaphoreType.DMA for _ in range(NSLOT)],
            [pltpu.SemaphoreType.DMA for _ in range(NSLOT)],
        ],
        compiler_params=pltpu.CompilerParams(use_tc_tiling_on_sc=False),
    )
    out = run(sent, word_table, tag_table)
    return out.reshape(b, l, odim)
